# baseline (speedup 1.0000x reference)
import functools

import jax
import jax.numpy as jnp
from jax import lax
from jax.experimental import pallas as pl
from jax.experimental.pallas import tpu as pltpu
from jax.experimental.pallas import tpu_sc as plsc

N_NODES = 10000
D = 128
N_EDGES = 320000
NPAD = 10240
NC = 2
NS = 16
NW = NC * NS
G = 128
NCH = 80
EPT = G * NCH
E_PAD = EPT * NW
DUMMY = NPAD - 1
RPS = NPAD // NS
GS = 4
GQ = G // GS
NROWS = E_PAD // G
NCH_A = 80
NCH_B = (2 * NCH) - NCH_A

_MESH = plsc.VectorSubcoreMesh(core_axis_name="c", subcore_axis_name="s")


@functools.partial(
    pl.kernel,
    out_type=jax.ShapeDtypeStruct((NC, NPAD, D), jnp.float32),
    mesh=_MESH,
    scratch_types=[
        pltpu.VMEM((NCH, G), jnp.int32),
        pltpu.VMEM((G, D), jnp.float32),
        pltpu.VMEM_SHARED((NPAD, D), jnp.float32),
        pltpu.SemaphoreType.DMA,
    ],
)
def _deg_kernel(dst_hbm, zeros16_hbm, ones_hbm, deg_hbm, dst_v, ones_v, deg_sp,
                sem):
    cid = lax.axis_index("c")
    sid = lax.axis_index("s")
    wid = sid * NC + cid
    pltpu.sync_copy(zeros16_hbm, deg_sp.at[pl.ds(sid * RPS, RPS)])
    pltpu.sync_copy(dst_hbm.at[pl.ds(wid * NCH, NCH)], dst_v)
    pltpu.sync_copy(ones_hbm, ones_v)
    plsc.subcore_barrier()
    descs = [
        pltpu.async_copy(ones_v, deg_sp.at[dst_v.at[k]], sem, add=True)
        for k in range(NCH)
    ]
    for d in descs:
        d.wait()
    plsc.subcore_barrier()
    pltpu.sync_copy(
        deg_sp.at[pl.ds(sid * RPS, RPS)],
        deg_hbm.at[cid, pl.ds(sid * RPS, RPS)],
    )


@functools.partial(
    pl.kernel,
    out_type=jax.ShapeDtypeStruct((NC, NPAD, D), jnp.float32),
    mesh=_MESH,
    scratch_types=[
        pltpu.VMEM((2, G), jnp.int32),
        pltpu.VMEM((2, G), jnp.int32),
        pltpu.VMEM((G, D), jnp.float32),
        pltpu.VMEM((G, D), jnp.float32),
        pltpu.VMEM_SHARED((NPAD, D), jnp.float32),
        pltpu.SemaphoreType.DMA,
        pltpu.SemaphoreType.DMA,
        pltpu.SemaphoreType.DMA,
        pltpu.SemaphoreType.DMA,
        pltpu.SemaphoreType.DMA,
        pltpu.SemaphoreType.DMA,
        pltpu.SemaphoreType.DMA,
        pltpu.SemaphoreType.DMA,
    ],
)
def _prop_kernel(g_hbm, src_hbm, dst_hbm, zrows_hbm, acc_hbm,
                 srcq, dstq, st0, st1, acc_sp,
                 sg0, sg1, ssi0, ssi1, sdi0, sdi1, ssc0, ssc1):
    cid = lax.axis_index("c")
    sid = lax.axis_index("s")
    st = (st0, st1)
    sg = (sg0, sg1)
    ssi = (ssi0, ssi1)
    sdi = (sdi0, sdi1)
    ssc = (ssc0, ssc1)

    def issue_gathers(slot):
        return [
            pltpu.async_copy(
                g_hbm.at[srcq.at[slot, pl.ds(h * GQ, GQ)]],
                st[slot].at[pl.ds(h * GQ, GQ)],
                sg[slot],
            )
            for h in range(GS)
        ]

    def pipeline(base, nch):
        pltpu.sync_copy(src_hbm.at[base], srcq.at[0])
        pltpu.sync_copy(dst_hbm.at[base], dstq.at[0])
        gd = [None, None]
        gd[0] = issue_gathers(0)
        sidesc = [None, None]
        didesc = [None, None]
        sidesc[1] = pltpu.async_copy(src_hbm.at[base + 1], srcq.at[1], ssi[1])
        didesc[1] = pltpu.async_copy(dst_hbm.at[base + 1], dstq.at[1], sdi[1])
        scdesc = [None, None]
        for k in range(nch):
            b = k & 1
            nb = b ^ 1
            if k + 1 < nch:
                sidesc[nb].wait()
                if k >= 1:
                    scdesc[nb].wait()
                    didesc[nb] = pltpu.async_copy(
                        dst_hbm.at[base + k + 1], dstq.at[nb], sdi[nb])
                gd[nb] = issue_gathers(nb)
            for d in gd[b]:
                d.wait()
            if k + 2 < nch:
                sidesc[b] = pltpu.async_copy(
                    src_hbm.at[base + k + 2], srcq.at[b], ssi[b])
            if k > 0:
                didesc[b].wait()
            scdesc[b] = pltpu.async_copy(
                st[b], acc_sp.at[dstq.at[b]], ssc[b], add=True)
        scdesc[(nch - 1) & 1].wait()
        if nch >= 2:
            scdesc[nch & 1].wait()

    pltpu.sync_copy(zrows_hbm, acc_sp.at[pl.ds(sid * RPS, RPS)])
    plsc.subcore_barrier()

    @pl.when(cid == 0)
    def _():
        pipeline(sid * NCH_A, NCH_A)

    @pl.when(cid == 1)
    def _():
        pipeline(NS * NCH_A + sid * NCH_B, NCH_B)

    plsc.subcore_barrier()
    pltpu.sync_copy(
        acc_sp.at[pl.ds(sid * RPS, RPS)],
        acc_hbm.at[cid, pl.ds(sid * RPS, RPS)],
    )


_R = 1024
_GRID = NPAD // _R


def _mm_scale_body(x_ref, w_ref, dega_ref, degb_ref, g_ref, dinv_ref):
    deg = dega_ref[:, :1] + degb_ref[:, :1] + 1.0
    dinv = lax.rsqrt(deg)
    h = jnp.dot(x_ref[...], w_ref[...], preferred_element_type=jnp.float32)
    g_ref[...] = h * dinv
    dinv_ref[...] = dinv


def _mid_body(acca_ref, accb_ref, g1_ref, dinv_ref, b1_ref, w2_ref, g2_ref):
    dinv = dinv_ref[...]
    pre = (acca_ref[...] + accb_ref[...] + g1_ref[...]) * dinv + b1_ref[...]
    z = 0.5 * pre * (1.0 + lax.erf(pre * (2.0 ** -0.5)))
    g2_ref[...] = jnp.dot(z, w2_ref[...], preferred_element_type=jnp.float32) * dinv


def _out_body(acca_ref, accb_ref, g2_ref, dinv_ref, b2_ref, out_ref):
    out_ref[...] = (
        (acca_ref[...] + accb_ref[...] + g2_ref[...]) * dinv_ref[...] + b2_ref[...]
    )


def _row_spec(width):
    return pl.BlockSpec((_R, width), lambda i: (i, 0))


def _full_spec(shape):
    return pl.BlockSpec(shape, lambda i: (0,) * len(shape))


_mm_scale = pl.pallas_call(
    _mm_scale_body,
    grid=(_GRID,),
    in_specs=[_row_spec(D), _full_spec((D, D)), _row_spec(D), _row_spec(D)],
    out_specs=[_row_spec(D), _row_spec(1)],
    out_shape=[
        jax.ShapeDtypeStruct((NPAD, D), jnp.float32),
        jax.ShapeDtypeStruct((NPAD, 1), jnp.float32),
    ],
)

_mid = pl.pallas_call(
    _mid_body,
    grid=(_GRID,),
    in_specs=[_row_spec(D), _row_spec(D), _row_spec(D), _row_spec(1),
              _full_spec((1, D)), _full_spec((D, D))],
    out_specs=_row_spec(D),
    out_shape=jax.ShapeDtypeStruct((NPAD, D), jnp.float32),
)

_outk = pl.pallas_call(
    _out_body,
    grid=(_GRID,),
    in_specs=[_row_spec(D), _row_spec(D), _row_spec(D), _row_spec(1),
              _full_spec((1, D))],
    out_specs=_row_spec(D),
    out_shape=jax.ShapeDtypeStruct((NPAD, D), jnp.float32),
)


@jax.jit
def kernel(x, edge_index, W1, b1, W2, b2):
    e = edge_index.astype(jnp.int32)
    pad = (jnp.arange(E_PAD - N_EDGES, dtype=jnp.int32) % (NPAD - N_NODES)
           ) + N_NODES
    src = jnp.concatenate([e[0], pad]).reshape(NROWS, G)
    dst = jnp.concatenate([e[1], pad]).reshape(NROWS, G)
    x_pad = jnp.pad(x, ((0, NPAD - N_NODES), (0, 0)))

    ones = jnp.ones((G, D), jnp.float32)
    zrows = jnp.zeros((RPS, D), jnp.float32)

    deg = _deg_kernel(dst, zrows, ones)
    g1, dinv = _mm_scale(x_pad, W1, deg[0], deg[1])
    acc1 = _prop_kernel(g1, src, dst, zrows)
    g2 = _mid(acc1[0], acc1[1], g1, dinv, b1.reshape(1, D), W2)
    acc2 = _prop_kernel(g2, src, dst, zrows)
    out = _outk(acc2[0], acc2[1], g2, dinv, b2.reshape(1, D))
    return out[:N_NODES]


# speedup vs baseline: 1.0065x; 1.0065x over previous
import functools

import jax
import jax.numpy as jnp
from jax import lax
from jax.experimental import pallas as pl
from jax.experimental.pallas import tpu as pltpu
from jax.experimental.pallas import tpu_sc as plsc

N_NODES = 10000
D = 128
N_EDGES = 320000
NPAD = 10240
NC = 2
NS = 16
NW = NC * NS
G = 128
NCH = 80
EPT = G * NCH
E_PAD = EPT * NW
DUMMY = NPAD - 1
RPS = NPAD // NS
GS = 2
GQ = G // GS
NROWS = E_PAD // G
NCH_A = 80
NCH_B = (2 * NCH) - NCH_A

_MESH = plsc.VectorSubcoreMesh(core_axis_name="c", subcore_axis_name="s")


@functools.partial(
    pl.kernel,
    out_type=jax.ShapeDtypeStruct((NC, NPAD, D), jnp.float32),
    mesh=_MESH,
    scratch_types=[
        pltpu.VMEM((NCH, G), jnp.int32),
        pltpu.VMEM((G, D), jnp.float32),
        pltpu.VMEM_SHARED((NPAD, D), jnp.float32),
        pltpu.SemaphoreType.DMA,
    ],
)
def _deg_kernel(dst_hbm, zeros_hbm, ones_hbm, deg_hbm, dst_v, ones_v, deg_sp,
                sem):
    cid = lax.axis_index("c")
    sid = lax.axis_index("s")
    wid = sid * NC + cid
    pltpu.sync_copy(zeros_hbm, deg_sp.at[pl.ds(sid * RPS, RPS)])
    pltpu.sync_copy(dst_hbm.at[pl.ds(wid * NCH, NCH)], dst_v)
    pltpu.sync_copy(ones_hbm, ones_v)
    plsc.subcore_barrier()
    descs = [
        pltpu.async_copy(ones_v, deg_sp.at[dst_v.at[k]], sem, add=True)
        for k in range(NCH)
    ]
    for d in descs:
        d.wait()
    plsc.subcore_barrier()
    pltpu.sync_copy(
        deg_sp.at[pl.ds(sid * RPS, RPS)],
        deg_hbm.at[cid, pl.ds(sid * RPS, RPS)],
    )


@functools.partial(
    pl.kernel,
    out_type=jax.ShapeDtypeStruct((NC, NPAD, D), jnp.float32),
    mesh=_MESH,
    scratch_types=[
        pltpu.VMEM((2, G), jnp.int32),
        pltpu.VMEM((2, G), jnp.int32),
        pltpu.VMEM((G, D), jnp.float32),
        pltpu.VMEM((G, D), jnp.float32),
        pltpu.VMEM_SHARED((NPAD, D), jnp.float32),
        pltpu.SemaphoreType.DMA,
        pltpu.SemaphoreType.DMA,
        pltpu.SemaphoreType.DMA,
        pltpu.SemaphoreType.DMA,
        pltpu.SemaphoreType.DMA,
        pltpu.SemaphoreType.DMA,
        pltpu.SemaphoreType.DMA,
        pltpu.SemaphoreType.DMA,
    ],
)
def _prop_kernel(g_hbm, src_hbm, dst_hbm, zrows_hbm, acc_hbm,
                 srcq, dstq, st0, st1, acc_sp,
                 sg0, sg1, ssi0, ssi1, sdi0, sdi1, ssc0, ssc1):
    cid = lax.axis_index("c")
    sid = lax.axis_index("s")
    st = (st0, st1)
    sg = (sg0, sg1)
    ssi = (ssi0, ssi1)
    sdi = (sdi0, sdi1)
    ssc = (ssc0, ssc1)

    def issue_gathers(slot):
        return [
            pltpu.async_copy(
                g_hbm.at[srcq.at[slot, pl.ds(h * GQ, GQ)]],
                st[slot].at[pl.ds(h * GQ, GQ)],
                sg[slot],
            )
            for h in range(GS)
        ]

    def pipeline(base, nch):
        pltpu.sync_copy(src_hbm.at[base], srcq.at[0])
        pltpu.sync_copy(dst_hbm.at[base], dstq.at[0])
        gd = [None, None]
        gd[0] = issue_gathers(0)
        sidesc = [None, None]
        didesc = [None, None]
        sidesc[1] = pltpu.async_copy(src_hbm.at[base + 1], srcq.at[1], ssi[1])
        didesc[1] = pltpu.async_copy(dst_hbm.at[base + 1], dstq.at[1], sdi[1])
        scdesc = [None, None]
        for k in range(nch):
            b = k & 1
            nb = b ^ 1
            if k + 1 < nch:
                sidesc[nb].wait()
                if k >= 1:
                    scdesc[nb].wait()
                    didesc[nb] = pltpu.async_copy(
                        dst_hbm.at[base + k + 1], dstq.at[nb], sdi[nb])
                gd[nb] = issue_gathers(nb)
            for d in gd[b]:
                d.wait()
            if k + 2 < nch:
                sidesc[b] = pltpu.async_copy(
                    src_hbm.at[base + k + 2], srcq.at[b], ssi[b])
            if k > 0:
                didesc[b].wait()
            scdesc[b] = pltpu.async_copy(
                st[b], acc_sp.at[dstq.at[b]], ssc[b], add=True)
        scdesc[(nch - 1) & 1].wait()
        if nch >= 2:
            scdesc[nch & 1].wait()

    pltpu.sync_copy(zrows_hbm, acc_sp.at[pl.ds(sid * RPS, RPS)])
    plsc.subcore_barrier()

    @pl.when(cid == 0)
    def _():
        pipeline(sid * NCH_A, NCH_A)

    @pl.when(cid == 1)
    def _():
        pipeline(NS * NCH_A + sid * NCH_B, NCH_B)

    plsc.subcore_barrier()
    pltpu.sync_copy(
        acc_sp.at[pl.ds(sid * RPS, RPS)],
        acc_hbm.at[cid, pl.ds(sid * RPS, RPS)],
    )


_R = 1024
_GRID = NPAD // _R


def _mm_scale_body(x_ref, w_ref, dega_ref, degb_ref, g_ref, dinv_ref):
    deg = dega_ref[:, :1] + degb_ref[:, :1] + 1.0
    dinv = lax.rsqrt(deg)
    h = jnp.dot(x_ref[...], w_ref[...], preferred_element_type=jnp.float32)
    g_ref[...] = h * dinv
    dinv_ref[...] = dinv


def _mid_body(acca_ref, accb_ref, g1_ref, dinv_ref, b1_ref, w2_ref, g2_ref):
    dinv = dinv_ref[...]
    pre = (acca_ref[...] + accb_ref[...] + g1_ref[...]) * dinv + b1_ref[...]
    z = 0.5 * pre * (1.0 + lax.erf(pre * (2.0 ** -0.5)))
    g2_ref[...] = jnp.dot(z, w2_ref[...], preferred_element_type=jnp.float32) * dinv


def _out_body(acca_ref, accb_ref, g2_ref, dinv_ref, b2_ref, out_ref):
    out_ref[...] = (
        (acca_ref[...] + accb_ref[...] + g2_ref[...]) * dinv_ref[...] + b2_ref[...]
    )


def _row_spec(width):
    return pl.BlockSpec((_R, width), lambda i: (i, 0))


def _full_spec(shape):
    return pl.BlockSpec(shape, lambda i: (0,) * len(shape))


_mm_scale = pl.pallas_call(
    _mm_scale_body,
    grid=(_GRID,),
    in_specs=[_row_spec(D), _full_spec((D, D)), _row_spec(D), _row_spec(D)],
    out_specs=[_row_spec(D), _row_spec(1)],
    out_shape=[
        jax.ShapeDtypeStruct((NPAD, D), jnp.float32),
        jax.ShapeDtypeStruct((NPAD, 1), jnp.float32),
    ],
)

_mid = pl.pallas_call(
    _mid_body,
    grid=(_GRID,),
    in_specs=[_row_spec(D), _row_spec(D), _row_spec(D), _row_spec(1),
              _full_spec((1, D)), _full_spec((D, D))],
    out_specs=_row_spec(D),
    out_shape=jax.ShapeDtypeStruct((NPAD, D), jnp.float32),
)

_outk = pl.pallas_call(
    _out_body,
    grid=(_GRID,),
    in_specs=[_row_spec(D), _row_spec(D), _row_spec(D), _row_spec(1),
              _full_spec((1, D))],
    out_specs=_row_spec(D),
    out_shape=jax.ShapeDtypeStruct((NPAD, D), jnp.float32),
)


@jax.jit
def kernel(x, edge_index, W1, b1, W2, b2):
    e = edge_index.astype(jnp.int32)
    pad = (jnp.arange(E_PAD - N_EDGES, dtype=jnp.int32) % (NPAD - N_NODES)
           ) + N_NODES
    src = jnp.concatenate([e[0], pad]).reshape(NROWS, G)
    dst = jnp.concatenate([e[1], pad]).reshape(NROWS, G)
    x_pad = jnp.pad(x, ((0, NPAD - N_NODES), (0, 0)))

    zrows = jnp.zeros((RPS, D), jnp.float32)

    ones = jnp.ones((G, D), jnp.float32)
    deg = _deg_kernel(dst, zrows, ones)
    g1, dinv = _mm_scale(x_pad, W1, deg[0], deg[1])
    acc1 = _prop_kernel(g1, src, dst, zrows)
    g2 = _mid(acc1[0], acc1[1], g1, dinv, b1.reshape(1, D), W2)
    acc2 = _prop_kernel(g2, src, dst, zrows)
    out = _outk(acc2[0], acc2[1], g2, dinv, b2.reshape(1, D))
    return out[:N_NODES]
